# baseline (speedup 1.0000x reference)
import jax
import jax.numpy as jnp
from jax import lax
from jax.experimental import pallas as pl
from jax.experimental.pallas import tpu as pltpu
from jax.experimental.pallas import tpu_sc as plsc

_NC, _NS, _L = 2, 16, 16
_NW = _NC * _NS
_B, _H, _D = 4096, 200, 128
_RPW = _B // _NW
_CHUNK = 100
_NCHUNK = _H // _CHUNK
_DV = _D // _L
_UNROLL = 8


def _sc_body(x_hbm, f_hbm, o_hbm, idx_v, rows0, rows1, out_v, semA, semB):
    wid = lax.axis_index("s") * _NC + lax.axis_index("c")
    pltpu.sync_copy(x_hbm.at[wid], idx_v)

    def issue(r, rows, sem):
        d = []
        for j in range(_NCHUNK):
            d.append(pltpu.async_copy(
                f_hbm.at[idx_v.at[r * _NCHUNK + j]],
                rows.at[pl.ds(j * _CHUNK, _CHUNK)], sem))
        return d

    def drain(rows, sem):
        pltpu.make_async_copy(f_hbm.at[pl.ds(0, _H)], rows, sem).wait()

    def reduce_into(rows, r):
        def h_body(h, accs):
            base = h * _UNROLL
            for u in range(_UNROLL):
                accs = tuple(accs[c] + rows[base + u, pl.ds(c * _L, _L)]
                             for c in range(_DV))
            return accs

        accs = tuple(jnp.zeros((_L,), jnp.float32) for _ in range(_DV))
        accs = lax.fori_loop(0, _H // _UNROLL, h_body, accs)
        for c in range(_DV):
            out_v[r, pl.ds(c * _L, _L)] = accs[c]

    issue(0, rows0, semA)

    @pl.loop(0, _RPW - 2, step=2)
    def _pair(r):
        dB = issue(r + 1, rows1, semB)
        drain(rows0, semA)
        reduce_into(rows0, r)
        issue(r + 2, rows0, semA)
        for d in dB:
            d.wait()
        reduce_into(rows1, r + 1)

    dB = issue(_RPW - 1, rows1, semB)
    drain(rows0, semA)
    reduce_into(rows0, _RPW - 2)
    for d in dB:
        d.wait()
    reduce_into(rows1, _RPW - 1)

    pltpu.sync_copy(out_v, o_hbm.at[pl.ds(wid * _RPW, _RPW)])


def kernel(x, F):
    x3 = x.reshape(_NW, _RPW * _NCHUNK, _CHUNK)
    mesh = plsc.VectorSubcoreMesh(
        core_axis_name="c", subcore_axis_name="s",
        num_cores=_NC, num_subcores=_NS,
    )
    run = pl.kernel(
        _sc_body,
        out_type=jax.ShapeDtypeStruct((_B, _D), jnp.float32),
        mesh=mesh,
        scratch_types=[
            pltpu.VMEM((_RPW * _NCHUNK, _CHUNK), jnp.int32),
            pltpu.VMEM((_H, _D), jnp.float32),
            pltpu.VMEM((_H, _D), jnp.float32),
            pltpu.VMEM((_RPW, _D), jnp.float32),
            pltpu.SemaphoreType.DMA,
            pltpu.SemaphoreType.DMA,
        ],
    )
    return run(x3, F)


# speedup vs baseline: 1.2179x; 1.2179x over previous
import jax
import jax.numpy as jnp
from jax import lax
from jax.experimental import pallas as pl
from jax.experimental.pallas import tpu as pltpu
from jax.experimental.pallas import tpu_sc as plsc

_NC, _NS, _L = 2, 16, 16
_NW = _NC * _NS
_B, _H, _D = 4096, 200, 128
_RPW = _B // _NW
_CHUNK = 100
_NCHUNK = _H // _CHUNK
_NCH_TOT = _RPW * _NCHUNK
_DV = _D // _L
_UNROLL = 4
_NBUF = 4


def _sc_body(x_hbm, f_hbm, o_hbm, idx_v, rows_v, out_v, s0, s1, s2, s3):
    sems = (s0, s1, s2, s3)
    wid = lax.axis_index("s") * _NC + lax.axis_index("c")
    pltpu.sync_copy(x_hbm.at[wid], idx_v)

    def issue(c, k):
        cc = jnp.minimum(c, _NCH_TOT - 1)
        pltpu.async_copy(f_hbm.at[idx_v.at[cc]], rows_v.at[k], sems[k])

    def drain(k):
        pltpu.make_async_copy(f_hbm.at[idx_v.at[0]], rows_v.at[k],
                              sems[k]).wait()

    def accumulate(k, accs):
        def h_body(h, accs):
            base = h * _UNROLL
            for u in range(_UNROLL):
                accs = tuple(accs[c] + rows_v[k, base + u, pl.ds(c * _L, _L)]
                             for c in range(_DV))
            return accs
        return lax.fori_loop(0, _CHUNK // _UNROLL, h_body, accs)

    def zeros():
        return tuple(jnp.zeros((_L,), jnp.float32) for _ in range(_DV))

    for k in range(_NBUF - 1):
        issue(k, k)

    @pl.loop(0, _NCH_TOT, step=_NBUF)
    def _macro(t):
        accs = zeros()
        for k in range(_NBUF):
            issue(t + k + _NBUF - 1, (k + _NBUF - 1) % _NBUF)
            drain(k)
            accs = accumulate(k, accs)
            if k % _NCHUNK == _NCHUNK - 1:
                r = (t >> 1) + (k >> 1)
                for c in range(_DV):
                    out_v[r, pl.ds(c * _L, _L)] = accs[c]
                accs = zeros()

    for k in range(_NBUF - 1):
        drain(k)

    pltpu.sync_copy(out_v, o_hbm.at[pl.ds(wid * _RPW, _RPW)])


def kernel(x, F):
    x3 = x.reshape(_NW, _NCH_TOT, _CHUNK)
    mesh = plsc.VectorSubcoreMesh(
        core_axis_name="c", subcore_axis_name="s",
        num_cores=_NC, num_subcores=_NS,
    )
    run = pl.kernel(
        _sc_body,
        out_type=jax.ShapeDtypeStruct((_B, _D), jnp.float32),
        mesh=mesh,
        scratch_types=[
            pltpu.VMEM((_NCH_TOT, _CHUNK), jnp.int32),
            pltpu.VMEM((_NBUF, _CHUNK, _D), jnp.float32),
            pltpu.VMEM((_RPW, _D), jnp.float32),
            pltpu.SemaphoreType.DMA,
            pltpu.SemaphoreType.DMA,
            pltpu.SemaphoreType.DMA,
            pltpu.SemaphoreType.DMA,
        ],
    )
    return run(x3, F)


# speedup vs baseline: 1.2321x; 1.0117x over previous
import jax
import jax.numpy as jnp
from jax import lax
from jax.experimental import pallas as pl
from jax.experimental.pallas import tpu as pltpu
from jax.experimental.pallas import tpu_sc as plsc

_NC, _NS, _L = 2, 16, 16
_NW = _NC * _NS
_B, _H, _D = 4096, 200, 128
_RPW = _B // _NW
_CHUNK = 50
_NCHUNK = _H // _CHUNK
_NCH_TOT = _RPW * _NCHUNK
_DV = _D // _L
_UNROLL = 5
_NBUF = 8
_OROWS = 32


def _sc_body(x_hbm, f_hbm, o_hbm, idx_v, rows_v, out_v, *sems):
    wid = lax.axis_index("s") * _NC + lax.axis_index("c")
    pltpu.sync_copy(x_hbm.at[wid], idx_v)

    def issue(c, k):
        cc = jnp.minimum(c, _NCH_TOT - 1)
        pltpu.async_copy(f_hbm.at[idx_v.at[cc]], rows_v.at[k], sems[k])

    def drain(k):
        pltpu.make_async_copy(f_hbm.at[idx_v.at[0]], rows_v.at[k],
                              sems[k]).wait()

    def accumulate(k, accs):
        def h_body(h, accs):
            base = h * _UNROLL
            for u in range(_UNROLL):
                accs = tuple(accs[c] + rows_v[k, base + u, pl.ds(c * _L, _L)]
                             for c in range(_DV))
            return accs
        return lax.fori_loop(0, _CHUNK // _UNROLL, h_body, accs)

    def zeros():
        return tuple(jnp.zeros((_L,), jnp.float32) for _ in range(_DV))

    for k in range(_NBUF - 1):
        issue(k, k)

    @pl.loop(0, _NCH_TOT, step=_NBUF)
    def _macro(t):
        accs = zeros()
        for k in range(_NBUF):
            issue(t + k + _NBUF - 1, (k + _NBUF - 1) % _NBUF)
            drain(k)
            accs = accumulate(k, accs)
            if k % _NCHUNK == _NCHUNK - 1:
                r = (t // _NCHUNK) + (k // _NCHUNK)
                for c in range(_DV):
                    out_v[r & (_OROWS - 1), pl.ds(c * _L, _L)] = accs[c]
                accs = zeros()

        @pl.when((t & (_OROWS * _NCHUNK - 1)) == _OROWS * _NCHUNK - _NBUF)
        def _flush():
            q = t // (_OROWS * _NCHUNK)
            pltpu.sync_copy(out_v,
                            o_hbm.at[pl.ds(wid * _RPW + q * _OROWS, _OROWS)])

    for k in range(_NBUF - 1):
        drain(k)


def kernel(x, F):
    x3 = x.reshape(_NW, _NCH_TOT, _CHUNK)
    mesh = plsc.VectorSubcoreMesh(
        core_axis_name="c", subcore_axis_name="s",
        num_cores=_NC, num_subcores=_NS,
    )
    run = pl.kernel(
        _sc_body,
        out_type=jax.ShapeDtypeStruct((_B, _D), jnp.float32),
        mesh=mesh,
        scratch_types=[
            pltpu.VMEM((_NCH_TOT, _CHUNK), jnp.int32),
            pltpu.VMEM((_NBUF, _CHUNK, _D), jnp.float32),
            pltpu.VMEM((_OROWS, _D), jnp.float32),
        ] + [pltpu.SemaphoreType.DMA] * _NBUF,
    )
    return run(x3, F)
